# baseline (speedup 1.0000x reference)
import functools

import jax
import jax.numpy as jnp
from jax import lax
from jax.experimental import pallas as pl
from jax.experimental.pallas import tpu as pltpu
from jax.experimental.pallas import tpu_sc as plsc

_V = 1000000
_D = 32
_B = 16384
_NC = 2
_NS = 16
_NW = _NC * _NS
_BPW = _B // _NW
_L = 16
_GRPS = _BPW // _L


def _make_sc_kernel():
    mesh = plsc.VectorSubcoreMesh(
        core_axis_name="c", subcore_axis_name="s",
        num_cores=_NC, num_subcores=_NS)

    @functools.partial(
        pl.kernel,
        out_type=jax.ShapeDtypeStruct((_B,), jnp.float32),
        mesh=mesh,
        compiler_params=pltpu.CompilerParams(
            needs_layout_passes=False,
            use_tc_tiling_on_sc=False,
        ),
        scratch_types=[
            pltpu.VMEM((_BPW,), jnp.int32),
            pltpu.VMEM((_BPW,), jnp.int32),
            pltpu.VMEM((_BPW, _D), jnp.float32),
            pltpu.VMEM((_BPW, _D), jnp.float32),
            pltpu.VMEM((_BPW,), jnp.float32),
            pltpu.VMEM((_BPW,), jnp.float32),
            pltpu.VMEM((_BPW,), jnp.float32),
            pltpu.SemaphoreType.DMA,
            pltpu.SemaphoreType.DMA,
            pltpu.SemaphoreType.DMA,
            pltpu.SemaphoreType.DMA,
        ],
    )
    def sc_kernel(ids1_hbm, ids2_hbm, w1_hbm, b1_hbm, w2_hbm, b2_hbm,
                  out_hbm, ids1_v, ids2_v, rows1_v, rows2_v,
                  bias1_v, bias2_v, out_v, sem1, sem2, sem3, sem4):
        wid = lax.axis_index("s") * _NC + lax.axis_index("c")
        base = wid * _BPW

        pltpu.sync_copy(ids1_hbm.at[pl.ds(base, _BPW)], ids1_v)
        pltpu.sync_copy(ids2_hbm.at[pl.ds(base, _BPW)], ids2_v)

        c1 = pltpu.async_copy(w1_hbm.at[ids1_v], rows1_v, sem1)
        c2 = pltpu.async_copy(w2_hbm.at[ids2_v], rows2_v, sem2)
        c3 = pltpu.async_copy(b1_hbm.at[ids1_v], bias1_v, sem3)
        c4 = pltpu.async_copy(b2_hbm.at[ids2_v], bias2_v, sem4)
        c3.wait()
        c4.wait()
        c1.wait()
        c2.wait()

        def group(g, carry):
            row0 = g * _L
            rid = row0 + lax.iota(jnp.int32, _L)
            acc = bias1_v[pl.ds(row0, _L)] + bias2_v[pl.ds(row0, _L)]
            for d in range(_D):
                cid = jnp.full((_L,), d, jnp.int32)
                a = plsc.load_gather(rows1_v, [rid, cid])
                b = plsc.load_gather(rows2_v, [rid, cid])
                acc = acc + a * b
            out_v[pl.ds(row0, _L)] = acc
            return carry

        lax.fori_loop(0, _GRPS, group, 0)

        pltpu.sync_copy(out_v, out_hbm.at[pl.ds(base, _BPW)])

    return sc_kernel


_SC_KERNEL = _make_sc_kernel()


def kernel(ids1, ids2, W1, b1, W2, b2):
    return _SC_KERNEL(ids1.astype(jnp.int32), ids2.astype(jnp.int32),
                      W1, b1, W2, b2)
